# Initial kernel scaffold; baseline (speedup 1.0000x reference)
import jax
import jax.numpy as jnp
from jax.experimental import pallas as pl


def kernel(x, edge_index, edge_attr, W1, b1, W2, b2, W3, b3, Wf, bf):
    raise NotImplementedError("write your pallas kernel here")


# speedup vs baseline: 7.8163x; 7.8163x over previous
import functools

import jax
import jax.numpy as jnp
from jax import lax
from jax.experimental import pallas as pl
from jax.experimental.pallas import tpu as pltpu
from jax.experimental.pallas import tpu_sc as plsc

N = 10000
E = 320000
D = 128
NC = 2
NS = 16
NW = NC * NS
EPW = E // NW
CH = 80
NCHUNK = EPW // CH
NP = 10240
RPT = NP // NS
ZR = 128
LANES = 16

_mesh = functools.partial(
    plsc.VectorSubcoreMesh, core_axis_name="c", subcore_axis_name="s",
    num_cores=NC, num_subcores=NS)


def _worker_ids():
    cid = lax.axis_index("c")
    sid = lax.axis_index("s")
    return cid, sid, cid * NS + sid


def _deg_body(col_hbm, ew_hbm, out_hbm, cbuf, wbuf, pbuf, zbuf, deg_sh, sem):
    cid, sid, wid = _worker_ids()
    zvec = jnp.zeros((LANES,), jnp.float32)

    def _z(i, _):
        zbuf[i, pl.ds(0, LANES)] = zvec
        return 0
    lax.fori_loop(0, ZR, _z, 0)

    for r in range(RPT // ZR):
        pltpu.sync_copy(zbuf, deg_sh.at[pl.ds(sid * RPT + r * ZR, ZR)])
    plsc.subcore_barrier()

    base0 = wid * EPW

    def _chunk(i, _):
        base = base0 + i * CH
        pltpu.sync_copy(col_hbm.at[pl.ds(base, CH)], cbuf)
        pltpu.sync_copy(ew_hbm.at[pl.ds(base, CH)], wbuf)
        for g in range(CH // LANES):
            wv = wbuf[pl.ds(g * LANES, LANES)]
            for l in range(LANES):
                r = g * LANES + l
                pbuf[r, pl.ds(0, LANES)] = pbuf[r, pl.ds(0, LANES)] * 0.0 + wv[l]
        pltpu.sync_copy(pbuf, deg_sh.at[cbuf], add=True)
        return 0
    lax.fori_loop(0, NCHUNK, _chunk, 0)

    plsc.subcore_barrier()
    for r in range(RPT // ZR):
        off = sid * RPT + r * ZR
        pltpu.sync_copy(deg_sh.at[pl.ds(off, ZR)], zbuf)
        pltpu.sync_copy(zbuf, out_hbm.at[cid, pl.ds(off, ZR)])


def _sc_degree(col, ew):
    return pl.kernel(
        _deg_body,
        out_type=jax.ShapeDtypeStruct((NC, NP, LANES), jnp.float32),
        mesh=_mesh(),
        scratch_types=[
            pltpu.VMEM((CH,), jnp.int32),
            pltpu.VMEM((CH,), jnp.float32),
            pltpu.VMEM((CH, LANES), jnp.float32),
            pltpu.VMEM((ZR, LANES), jnp.float32),
            pltpu.VMEM_SHARED((NP, LANES), jnp.float32),
            pltpu.SemaphoreType.DMA,
        ],
    )(col, ew)


def _agg_body(h_hbm, row_hbm, col_hbm, ew_hbm, out_hbm,
              rbuf, cbuf, wbuf, gbuf, zbuf, acc_sh, sem):
    cid, sid, wid = _worker_ids()
    zvec = jnp.zeros((LANES,), jnp.float32)

    def _z(i, _):
        for k in range(D // LANES):
            zbuf[i, pl.ds(k * LANES, LANES)] = zvec
        return 0
    lax.fori_loop(0, ZR, _z, 0)
    for r in range(RPT // ZR):
        pltpu.sync_copy(zbuf, acc_sh.at[pl.ds(sid * RPT + r * ZR, ZR)])
    plsc.subcore_barrier()

    base0 = wid * EPW

    def _chunk(i, _):
        base = base0 + i * CH
        pltpu.sync_copy(row_hbm.at[pl.ds(base, CH)], rbuf)
        pltpu.sync_copy(col_hbm.at[pl.ds(base, CH)], cbuf)
        pltpu.sync_copy(ew_hbm.at[pl.ds(base, CH)], wbuf)
        pltpu.async_copy(h_hbm.at[rbuf], gbuf, sem).wait()

        for g in range(CH // LANES):
            wv = wbuf[pl.ds(g * LANES, LANES)]
            for l in range(LANES):
                s = wv[l]
                e = g * LANES + l
                for k in range(D // LANES):
                    sl = pl.ds(k * LANES, LANES)
                    gbuf[e, sl] = gbuf[e, sl] * s

        pltpu.sync_copy(gbuf, acc_sh.at[cbuf], add=True)
        return 0
    lax.fori_loop(0, NCHUNK, _chunk, 0)

    plsc.subcore_barrier()
    for r in range(RPT // ZR):
        off = sid * RPT + r * ZR
        pltpu.sync_copy(acc_sh.at[pl.ds(off, ZR)], zbuf)
        pltpu.sync_copy(zbuf, out_hbm.at[cid, pl.ds(off, ZR)])


def _sc_aggregate(h, row, col, ew):
    return pl.kernel(
        _agg_body,
        out_type=jax.ShapeDtypeStruct((NC, NP, D), jnp.float32),
        mesh=_mesh(),
        scratch_types=[
            pltpu.VMEM((CH,), jnp.int32),
            pltpu.VMEM((CH,), jnp.int32),
            pltpu.VMEM((CH,), jnp.float32),
            pltpu.VMEM((CH, D), jnp.float32),
            pltpu.VMEM((ZR, D), jnp.float32),
            pltpu.VMEM_SHARED((NP, D), jnp.float32),
            pltpu.SemaphoreType.DMA,
        ],
    )(h, row, col, ew)


_R = 2000
_DOT = dict(precision=lax.Precision.HIGHEST, preferred_element_type=jnp.float32)


def _dinv_of(degp):
    deg = 1.0 + degp[0, :, 0] + degp[1, :, 0]
    return lax.rsqrt(deg)


def _prep_body(x_ref, w1_ref, degp_ref, t1_ref, h1_ref):
    t1 = lax.dot_general(x_ref[...], w1_ref[...], (((1,), (1,)), ((), ())), **_DOT)
    t1_ref[...] = t1
    dinv = _dinv_of(degp_ref[...])
    h1_ref[...] = dinv[:, None] * t1


def _tc_prep(x, W1, degp):
    return pl.pallas_call(
        _prep_body,
        grid=(N // _R,),
        in_specs=[
            pl.BlockSpec((_R, D), lambda i: (i, 0)),
            pl.BlockSpec((D, D), lambda i: (0, 0)),
            pl.BlockSpec((NC, _R, LANES), lambda i: (0, i, 0)),
        ],
        out_specs=[
            pl.BlockSpec((_R, D), lambda i: (i, 0)),
            pl.BlockSpec((_R, D), lambda i: (i, 0)),
        ],
        out_shape=[
            jax.ShapeDtypeStruct((N, D), jnp.float32),
            jax.ShapeDtypeStruct((N, D), jnp.float32),
        ],
    )(x, W1, degp)


def _post_body(degp_ref, accp_ref, t_ref, b_ref, w_ref, x_ref, tn_ref, hn_ref):
    dinv = _dinv_of(degp_ref[...])
    accp = accp_ref[...]
    acc = accp[0] + accp[1]
    t = t_ref[...]
    xl = jnp.maximum(dinv[:, None] * acc + (dinv * dinv)[:, None] * t + b_ref[...], 0.0)
    x_ref[...] = xl
    tn = lax.dot_general(xl, w_ref[...], (((1,), (1,)), ((), ())), **_DOT)
    tn_ref[...] = tn
    hn_ref[...] = dinv[:, None] * tn


def _tc_post(degp, accp, t, b, Wn):
    return pl.pallas_call(
        _post_body,
        grid=(N // _R,),
        in_specs=[
            pl.BlockSpec((NC, _R, LANES), lambda i: (0, i, 0)),
            pl.BlockSpec((NC, _R, D), lambda i: (0, i, 0)),
            pl.BlockSpec((_R, D), lambda i: (i, 0)),
            pl.BlockSpec((1, D), lambda i: (0, 0)),
            pl.BlockSpec((D, D), lambda i: (0, 0)),
        ],
        out_specs=[
            pl.BlockSpec((_R, D), lambda i: (i, 0)),
            pl.BlockSpec((_R, D), lambda i: (i, 0)),
            pl.BlockSpec((_R, D), lambda i: (i, 0)),
        ],
        out_shape=[
            jax.ShapeDtypeStruct((N, D), jnp.float32),
            jax.ShapeDtypeStruct((N, D), jnp.float32),
            jax.ShapeDtypeStruct((N, D), jnp.float32),
        ],
    )(degp, accp, t, b, Wn)


def _final_body(degp_ref, accp_ref, t_ref, b_ref, x0_ref, x1_ref, x2_ref,
                wf_ref, bf_ref, out_ref):
    dinv = _dinv_of(degp_ref[...])
    accp = accp_ref[...]
    acc = accp[0] + accp[1]
    x3 = jnp.maximum(
        dinv[:, None] * acc + (dinv * dinv)[:, None] * t_ref[...] + b_ref[...], 0.0)
    wf = wf_ref[...]
    dn = (((1,), (1,)), ((), ()))
    out = lax.dot_general(x0_ref[...], wf[:, 0:D], dn, **_DOT)
    out += lax.dot_general(x1_ref[...], wf[:, D:2 * D], dn, **_DOT)
    out += lax.dot_general(x2_ref[...], wf[:, 2 * D:3 * D], dn, **_DOT)
    out += lax.dot_general(x3, wf[:, 3 * D:4 * D], dn, **_DOT)
    out_ref[...] = out + bf_ref[...]


def _tc_final(degp, accp, t3, b3, x0, x1, x2, Wf, bf):
    return pl.pallas_call(
        _final_body,
        grid=(N // _R,),
        in_specs=[
            pl.BlockSpec((NC, _R, LANES), lambda i: (0, i, 0)),
            pl.BlockSpec((NC, _R, D), lambda i: (0, i, 0)),
            pl.BlockSpec((_R, D), lambda i: (i, 0)),
            pl.BlockSpec((1, D), lambda i: (0, 0)),
            pl.BlockSpec((_R, D), lambda i: (i, 0)),
            pl.BlockSpec((_R, D), lambda i: (i, 0)),
            pl.BlockSpec((_R, D), lambda i: (i, 0)),
            pl.BlockSpec((D, 4 * D), lambda i: (0, 0)),
            pl.BlockSpec((1, D), lambda i: (0, 0)),
        ],
        out_specs=pl.BlockSpec((_R, D), lambda i: (i, 0)),
        out_shape=jax.ShapeDtypeStruct((N, D), jnp.float32),
    )(degp, accp, t3, b3, x0, x1, x2, Wf, bf)


def kernel(x, edge_index, edge_attr, W1, b1, W2, b2, W3, b3, Wf, bf):
    row = edge_index[0]
    col = edge_index[1]
    degp = _sc_degree(col, edge_attr)
    t1, h1 = _tc_prep(x, W1, degp)
    acc1 = _sc_aggregate(h1, row, col, edge_attr)
    x1, t2, h2 = _tc_post(degp, acc1, t1, b1.reshape(1, D), W2)
    acc2 = _sc_aggregate(h2, row, col, edge_attr)
    x2, t3, h3 = _tc_post(degp, acc2, t2, b2.reshape(1, D), W3)
    acc3 = _sc_aggregate(h3, row, col, edge_attr)
    return _tc_final(degp, acc3, t3, b3.reshape(1, D), x, x1, x2, Wf,
                     bf.reshape(1, D))
